# baseline (speedup 1.0000x reference)
import functools

import jax
import jax.numpy as jnp
from jax import lax
from jax.experimental import pallas as pl
from jax.experimental.pallas import tpu as pltpu
from jax.experimental.pallas import tpu_sc as plsc

EMBED_DIM = 32
CHUNK = 1024
HALF = CHUNK // 2
NBUF = 2


def _sc_info():
    try:
        info = plsc.get_sparse_core_info()
        return info.num_cores, info.num_subcores
    except Exception:
        return 2, 16


@functools.cache
def _make_gather(B, V):
    NC, NS = _sc_info()
    NW = NC * NS
    NT = B // 128
    b_per_w = B // NW
    n_chunks = b_per_w // CHUNK
    n_pairs = n_chunks // NBUF
    assert B % (NW * CHUNK) == 0 and n_chunks % NBUF == 0

    mesh = plsc.VectorSubcoreMesh(core_axis_name="c", subcore_axis_name="s")

    @functools.partial(
        pl.kernel,
        mesh=mesh,
        out_type=jax.ShapeDtypeStruct((4 * NT * 8, 128), jnp.float32),
        scratch_types=[
            pltpu.VMEM((NBUF, CHUNK), jnp.int32),
            pltpu.VMEM((NBUF, CHUNK, EMBED_DIM), jnp.float32),
            pltpu.VMEM((NBUF, 4, HALF // 128 * 8, 128), jnp.float32),
            pltpu.VMEM((16 * 33,), jnp.float32),
            pltpu.VMEM((16 * 33,), jnp.float32),
            pltpu.VMEM((16 * 33,), jnp.float32),
            pltpu.VMEM((16 * 33,), jnp.float32),
            pltpu.SemaphoreType.DMA,
            pltpu.SemaphoreType.DMA,
            pltpu.SemaphoreType.DMA,
            pltpu.SemaphoreType.DMA,
        ],
        compiler_params=pltpu.CompilerParams(
            use_tc_tiling_on_sc=False, needs_layout_passes=False
        ),
    )
    def gather_kernel(
        idx_hbm, table_hbm, out_hbm, idx_v, rows_v, tbuf,
        stage_a, stage_b, stage_c, stage_d,
        gsem0, gsem1, wsem0, wsem1
    ):
        wid = lax.axis_index("s") * NC + lax.axis_index("c")
        wbase = wid * b_per_w
        gsems = (gsem0, gsem1)
        wsems = (wsem0, wsem1)
        stages = (stage_a, stage_b, stage_c, stage_d)
        iota33 = lax.iota(jnp.int32, 16) * 33
        trows = HALF // 128 * 8

        def fire(c, b):
            pltpu.sync_copy(idx_hbm.at[pl.ds(wbase + c * CHUNK, CHUNK)], idx_v.at[b])
            pltpu.async_copy(table_hbm.at[idx_v.at[b]], rows_v.at[b], gsems[b])

        def drain_gather(b):
            pltpu.make_async_copy(
                table_hbm.at[pl.ds(0, CHUNK)], rows_v.at[b], gsems[b]
            ).wait()

        def out_slice(c, h, tr):
            tbase = (wbase + c * CHUNK) // 128 + h * (HALF // 128)
            return out_hbm.at[pl.ds((tr * NT + tbase) * 8, trows)]

        def wait_write(c, h):
            for tr in range(4):
                pltpu.make_async_copy(
                    tbuf.at[h, tr], out_slice(c, h, tr), wsems[h]
                ).wait()

        def transpose_half(b, h):
            def body(gl0, carry):
                for p, stage in enumerate(stages):
                    gl = gl0 + p
                    jbase = h * HALF + gl * 16
                    r = (gl // 8) * 8
                    k0 = (gl % 8) * 16
                    for j in range(16):
                        for h16 in range(2):
                            v = rows_v[b, jbase + j, pl.ds(h16 * 16, 16)]
                            stage[pl.ds(j * 33 + h16 * 16, 16)] = v
                    for d in range(EMBED_DIM):
                        vt = plsc.load_gather(stage, [iota33 + d])
                        tbuf[h, d // 8, r + (d % 8), pl.ds(k0, 16)] = vt
                return carry

            lax.fori_loop(0, HALF // 64, lambda i, c: body(i * 4, c), 0)

        def write_half(c, h):
            for tr in range(4):
                pltpu.async_copy(tbuf.at[h, tr], out_slice(c, h, tr), wsems[h])

        def finish(c, b, first):
            drain_gather(b)
            for h in range(2):
                if first is None:
                    wait_write(c - 1, h)
                else:
                    @pl.when(jnp.logical_not(first))
                    def _():
                        wait_write(c - 1, h)
                transpose_half(b, h)
                write_half(c, h)

        def pair_body(g, carry):
            fire(g * 2, 0)

            @pl.when(g > 0)
            def _():
                finish(g * 2 - 1, 1, None)

            fire(g * 2 + 1, 1)
            finish(g * 2, 0, g == 0)
            return carry

        lax.fori_loop(0, n_pairs, pair_body, 0)

        finish(n_chunks - 1, 1, None)
        for h in range(2):
            wait_write(n_chunks - 1, h)

    return gather_kernel


def kernel(time, table):
    B = time.shape[0] * time.shape[1]
    NT = B // 128
    idx = time.reshape(B).astype(jnp.int32)
    o = _make_gather(B, table.shape[0])(idx, table)
    o = o.reshape(4, NT, 8, 128).transpose(0, 2, 1, 3).reshape(EMBED_DIM, B)
    return o.T


# speedup vs baseline: 2.7168x; 2.7168x over previous
import functools

import jax
import jax.numpy as jnp
from jax import lax
from jax.experimental import pallas as pl
from jax.experimental.pallas import tpu as pltpu
from jax.experimental.pallas import tpu_sc as plsc

EMBED_DIM = 32
CHUNK = 1024
HALF = CHUNK // 2
NBUF = 2


def _sc_info():
    try:
        info = plsc.get_sparse_core_info()
        return info.num_cores, info.num_subcores
    except Exception:
        return 2, 16


@functools.cache
def _make_gather(B, V):
    NC, NS = _sc_info()
    NW = NC * NS
    NT = B // 128
    b_per_w = B // NW
    n_chunks = b_per_w // CHUNK
    n_pairs = n_chunks // NBUF
    assert B % (NW * CHUNK) == 0 and n_chunks % NBUF == 0

    mesh = plsc.VectorSubcoreMesh(core_axis_name="c", subcore_axis_name="s")

    @functools.partial(
        pl.kernel,
        mesh=mesh,
        out_type=jax.ShapeDtypeStruct((4 * NT * 8, 128), jnp.float32),
        scratch_types=[
            pltpu.VMEM((NBUF, CHUNK), jnp.int32),
            pltpu.VMEM((NBUF, CHUNK, EMBED_DIM), jnp.float32),
            pltpu.VMEM((NBUF, 4, HALF // 128 * 8, 128), jnp.float32),
            pltpu.VMEM((16 * 33,), jnp.float32),
            pltpu.VMEM((16 * 33,), jnp.float32),
            pltpu.VMEM((16 * 33,), jnp.float32),
            pltpu.VMEM((16 * 33,), jnp.float32),
            pltpu.SemaphoreType.DMA,
            pltpu.SemaphoreType.DMA,
            pltpu.SemaphoreType.DMA,
            pltpu.SemaphoreType.DMA,
        ],
        compiler_params=pltpu.CompilerParams(
            use_tc_tiling_on_sc=False, needs_layout_passes=False
        ),
    )
    def gather_kernel(
        idx_hbm, table_hbm, out_hbm, idx_v, rows_v, tbuf,
        stage_a, stage_b, stage_c, stage_d,
        gsem0, gsem1, wsem0, wsem1
    ):
        wid = lax.axis_index("s") * NC + lax.axis_index("c")
        wbase = wid * b_per_w
        gsems = (gsem0, gsem1)
        wsems = (wsem0, wsem1)
        stages = (stage_a, stage_b, stage_c, stage_d)
        iota33 = lax.iota(jnp.int32, 16) * 33
        trows = HALF // 128 * 8

        def fire(c, b):
            pltpu.sync_copy(idx_hbm.at[pl.ds(wbase + c * CHUNK, CHUNK)], idx_v.at[b])
            pltpu.async_copy(table_hbm.at[idx_v.at[b]], rows_v.at[b], gsems[b])

        def drain_gather(b):
            pltpu.make_async_copy(
                table_hbm.at[pl.ds(0, CHUNK)], rows_v.at[b], gsems[b]
            ).wait()

        def out_slice(c, h, tr):
            tbase = (wbase + c * CHUNK) // 128 + h * (HALF // 128)
            return out_hbm.at[pl.ds((tr * NT + tbase) * 8, trows)]

        def wait_write(c, h):
            for tr in range(4):
                pltpu.make_async_copy(
                    tbuf.at[h, tr], out_slice(c, h, tr), wsems[h]
                ).wait()

        def transpose_half(b, h):
            def body(gl0, carry):
                for p, stage in enumerate(stages):
                    gl = gl0 + p
                    jbase = h * HALF + gl * 16
                    r = (gl // 8) * 8
                    k0 = (gl % 8) * 16

                    @plsc.parallel_loop(0, 16, 1, unroll=8)
                    def _stage1(j):
                        for h16 in range(2):
                            v = rows_v[b, jbase + j, pl.ds(h16 * 16, 16)]
                            stage[pl.ds(j * 33 + h16 * 16, 16)] = v

                    @plsc.parallel_loop(0, EMBED_DIM, 1, unroll=8)
                    def _stage2(d):
                        vt = plsc.load_gather(stage, [iota33 + d])
                        tbuf[h, d // 8, r + (d % 8), pl.ds(k0, 16)] = vt
                return carry

            lax.fori_loop(0, HALF // 64, lambda i, c: body(i * 4, c), 0)

        def write_half(c, h):
            for tr in range(4):
                pltpu.async_copy(tbuf.at[h, tr], out_slice(c, h, tr), wsems[h])

        def finish(c, b, first):
            drain_gather(b)
            for h in range(2):
                if first is None:
                    wait_write(c - 1, h)
                else:
                    @pl.when(jnp.logical_not(first))
                    def _():
                        wait_write(c - 1, h)
                transpose_half(b, h)
                write_half(c, h)

        def pair_body(g, carry):
            fire(g * 2, 0)

            @pl.when(g > 0)
            def _():
                finish(g * 2 - 1, 1, None)

            fire(g * 2 + 1, 1)
            finish(g * 2, 0, g == 0)
            return carry

        lax.fori_loop(0, n_pairs, pair_body, 0)

        finish(n_chunks - 1, 1, None)
        for h in range(2):
            wait_write(n_chunks - 1, h)

    return gather_kernel


def kernel(time, table):
    B = time.shape[0] * time.shape[1]
    NT = B // 128
    idx = time.reshape(B).astype(jnp.int32)
    o = _make_gather(B, table.shape[0])(idx, table)
    o = o.reshape(4, NT, 8, 128).transpose(0, 2, 1, 3).reshape(EMBED_DIM, B)
    return o.T


# speedup vs baseline: 2.8844x; 1.0617x over previous
import functools

import jax
import jax.numpy as jnp
from jax import lax
from jax.experimental import pallas as pl
from jax.experimental.pallas import tpu as pltpu
from jax.experimental.pallas import tpu_sc as plsc

EMBED_DIM = 32
CHUNK = 1024
HALF = CHUNK // 2
NBUF = 2


def _sc_info():
    try:
        info = plsc.get_sparse_core_info()
        return info.num_cores, info.num_subcores
    except Exception:
        return 2, 16


@functools.cache
def _make_gather(B, V):
    NC, NS = _sc_info()
    NW = NC * NS
    NT = B // 128
    b_per_w = B // NW
    n_chunks = b_per_w // CHUNK
    n_pairs = n_chunks // NBUF
    assert B % (NW * CHUNK) == 0 and n_chunks % NBUF == 0

    mesh = plsc.VectorSubcoreMesh(core_axis_name="c", subcore_axis_name="s")

    @functools.partial(
        pl.kernel,
        mesh=mesh,
        out_type=jax.ShapeDtypeStruct((4 * NT * 8, 128), jnp.float32),
        scratch_types=[
            pltpu.VMEM((NBUF, CHUNK), jnp.int32),
            pltpu.VMEM((NBUF, CHUNK, EMBED_DIM), jnp.float32),
            pltpu.VMEM((NBUF, 4, HALF // 128 * 8, 128), jnp.float32),
            pltpu.VMEM((16 * 33,), jnp.float32),
            pltpu.VMEM((16 * 33,), jnp.float32),
            pltpu.VMEM((16 * 33,), jnp.float32),
            pltpu.VMEM((16 * 33,), jnp.float32),
            pltpu.SemaphoreType.DMA,
            pltpu.SemaphoreType.DMA,
            pltpu.SemaphoreType.DMA,
            pltpu.SemaphoreType.DMA,
        ],
        compiler_params=pltpu.CompilerParams(
            use_tc_tiling_on_sc=False, needs_layout_passes=False
        ),
    )
    def gather_kernel(
        idx_hbm, table_hbm, out_hbm, idx_v, rows_v, tbuf,
        stage_a, stage_b, stage_c, stage_d,
        gsem0, gsem1, wsem0, wsem1
    ):
        wid = lax.axis_index("s") * NC + lax.axis_index("c")
        wbase = wid * b_per_w
        gsems = (gsem0, gsem1)
        wsems = (wsem0, wsem1)
        stages = (stage_a, stage_b, stage_c, stage_d)
        iota33 = lax.iota(jnp.int32, 16) * 33
        trows = HALF // 128 * 8

        def fire(c, b):
            pltpu.sync_copy(idx_hbm.at[pl.ds(wbase + c * CHUNK, CHUNK)], idx_v.at[b])
            pltpu.async_copy(table_hbm.at[idx_v.at[b]], rows_v.at[b], gsems[b])

        def drain_gather(b):
            pltpu.make_async_copy(
                table_hbm.at[pl.ds(0, CHUNK)], rows_v.at[b], gsems[b]
            ).wait()

        def out_slice(c, h, tr):
            tbase = (wbase + c * CHUNK) // 128 + h * (HALF // 128)
            return out_hbm.at[pl.ds((tr * NT + tbase) * 8, trows)]

        def wait_write(c, h):
            for tr in range(4):
                pltpu.make_async_copy(
                    tbuf.at[h, tr], out_slice(c, h, tr), wsems[h]
                ).wait()

        def transpose_half(b, h):
            def body(gl0, carry):
                for p, stage in enumerate(stages):
                    gl = gl0 + p
                    jbase = h * HALF + gl * 16
                    r = (gl // 8) * 8
                    k0 = (gl % 8) * 16

                    @plsc.parallel_loop(0, 16, 1, unroll=16)
                    def _stage1(j):
                        for h16 in range(2):
                            v = rows_v[b, jbase + j, pl.ds(h16 * 16, 16)]
                            stage[pl.ds(j * 33 + h16 * 16, 16)] = v

                    @plsc.parallel_loop(0, EMBED_DIM, 1, unroll=16)
                    def _stage2(d):
                        vt = plsc.load_gather(stage, [iota33 + d])
                        tbuf[h, d // 8, r + (d % 8), pl.ds(k0, 16)] = vt
                return carry

            lax.fori_loop(0, HALF // 64, lambda i, c: body(i * 4, c), 0)

        def write_half(c, h):
            for tr in range(4):
                pltpu.async_copy(tbuf.at[h, tr], out_slice(c, h, tr), wsems[h])

        def finish(c, b, first):
            drain_gather(b)
            for h in range(2):
                if first is None:
                    wait_write(c - 1, h)
                else:
                    @pl.when(jnp.logical_not(first))
                    def _():
                        wait_write(c - 1, h)
                transpose_half(b, h)
                write_half(c, h)

        def pair_body(g, carry):
            fire(g * 2, 0)

            @pl.when(g > 0)
            def _():
                finish(g * 2 - 1, 1, None)

            fire(g * 2 + 1, 1)
            finish(g * 2, 0, g == 0)
            return carry

        lax.fori_loop(0, n_pairs, pair_body, 0)

        finish(n_chunks - 1, 1, None)
        for h in range(2):
            wait_write(n_chunks - 1, h)

    return gather_kernel


def kernel(time, table):
    B = time.shape[0] * time.shape[1]
    NT = B // 128
    idx = time.reshape(B).astype(jnp.int32)
    o = _make_gather(B, table.shape[0])(idx, table)
    o = o.reshape(4, NT, 8, 128).transpose(0, 2, 1, 3).reshape(EMBED_DIM, B)
    return o.T


# speedup vs baseline: 3.4724x; 1.2039x over previous
import functools

import jax
import jax.numpy as jnp
from jax import lax
from jax.experimental import pallas as pl
from jax.experimental.pallas import tpu as pltpu
from jax.experimental.pallas import tpu_sc as plsc

EMBED_DIM = 32
CHUNK = 1024
HALF = CHUNK // 2
NBUF = 2


def _sc_info():
    try:
        info = plsc.get_sparse_core_info()
        return info.num_cores, info.num_subcores
    except Exception:
        return 2, 16


@functools.cache
def _make_gather(B, V):
    NC, NS = _sc_info()
    NW = NC * NS
    NT = B // 128
    b_per_w = B // NW
    n_chunks = b_per_w // CHUNK
    n_pairs = n_chunks // NBUF
    assert B % (NW * CHUNK) == 0 and n_chunks % NBUF == 0

    mesh = plsc.VectorSubcoreMesh(core_axis_name="c", subcore_axis_name="s")

    @functools.partial(
        pl.kernel,
        mesh=mesh,
        out_type=jax.ShapeDtypeStruct((4 * NT * 8, 128), jnp.float32),
        scratch_types=[
            pltpu.VMEM((NBUF, CHUNK), jnp.int32),
            pltpu.VMEM((NBUF, CHUNK, EMBED_DIM), jnp.float32),
            pltpu.VMEM((NBUF, 4, HALF // 128 * 8, 128), jnp.float32),
            pltpu.VMEM((16 * 33,), jnp.float32),
            pltpu.VMEM((16 * 33,), jnp.float32),
            pltpu.VMEM((16 * 33,), jnp.float32),
            pltpu.VMEM((16 * 33,), jnp.float32),
            pltpu.SemaphoreType.DMA,
            pltpu.SemaphoreType.DMA,
            pltpu.SemaphoreType.DMA,
            pltpu.SemaphoreType.DMA,
            pltpu.SemaphoreType.DMA,
            pltpu.SemaphoreType.DMA,
        ],
        compiler_params=pltpu.CompilerParams(
            use_tc_tiling_on_sc=False, needs_layout_passes=False
        ),
    )
    def gather_kernel(
        idx_hbm, table_hbm, out_hbm, idx_v, rows_v, tbuf,
        stage_a, stage_b, stage_c, stage_d,
        gsem0, gsem1, wsem0, wsem1, isem0, isem1
    ):
        wid = lax.axis_index("s") * NC + lax.axis_index("c")
        wbase = wid * b_per_w
        gsems = (gsem0, gsem1)
        wsems = (wsem0, wsem1)
        isems = (isem0, isem1)
        stages = (stage_a, stage_b, stage_c, stage_d)
        iota33 = lax.iota(jnp.int32, 16) * 33
        trows = HALF // 128 * 8

        def idx_slice(c):
            return idx_hbm.at[pl.ds(wbase + c * CHUNK, CHUNK)]

        def prefetch_idx(c, b):
            pltpu.async_copy(idx_slice(c), idx_v.at[b], isems[b])

        def fire(c, b):
            pltpu.make_async_copy(idx_slice(c), idx_v.at[b], isems[b]).wait()
            pltpu.async_copy(table_hbm.at[idx_v.at[b]], rows_v.at[b], gsems[b])

        def drain_gather(b):
            pltpu.make_async_copy(
                table_hbm.at[pl.ds(0, CHUNK)], rows_v.at[b], gsems[b]
            ).wait()

        def out_slice(c, h, tr):
            tbase = (wbase + c * CHUNK) // 128 + h * (HALF // 128)
            return out_hbm.at[pl.ds((tr * NT + tbase) * 8, trows)]

        def wait_write(c, h):
            for tr in range(4):
                pltpu.make_async_copy(
                    tbuf.at[h, tr], out_slice(c, h, tr), wsems[h]
                ).wait()

        def transpose_half(b, h):
            def body(gl0, carry):
                for p, stage in enumerate(stages):
                    gl = gl0 + p
                    jbase = h * HALF + gl * 16
                    r = (gl // 8) * 8
                    k0 = (gl % 8) * 16

                    @plsc.parallel_loop(0, 16, 1, unroll=16)
                    def _stage1(j):
                        for h16 in range(2):
                            v = rows_v[b, jbase + j, pl.ds(h16 * 16, 16)]
                            stage[pl.ds(j * 33 + h16 * 16, 16)] = v

                    @plsc.parallel_loop(0, EMBED_DIM, 1, unroll=16)
                    def _stage2(d):
                        vt = plsc.load_gather(stage, [iota33 + d])
                        tbuf[h, d // 8, r + (d % 8), pl.ds(k0, 16)] = vt
                return carry

            lax.fori_loop(0, HALF // 64, lambda i, c: body(i * 4, c), 0)

        def write_half(c, h):
            for tr in range(4):
                pltpu.async_copy(tbuf.at[h, tr], out_slice(c, h, tr), wsems[h])

        def finish(c, b, first, pf):
            drain_gather(b)
            if pf is True:
                prefetch_idx(c + 2, b)
            elif pf is not False:
                @pl.when(pf)
                def _():
                    prefetch_idx(c + 2, b)
            for h in range(2):
                if first is None:
                    wait_write(c - 1, h)
                else:
                    @pl.when(jnp.logical_not(first))
                    def _():
                        wait_write(c - 1, h)
                transpose_half(b, h)
                write_half(c, h)

        def pair_body(g, carry):
            fire(g * 2, 0)

            @pl.when(g > 0)
            def _():
                finish(g * 2 - 1, 1, None, True)

            fire(g * 2 + 1, 1)
            finish(g * 2, 0, g == 0, g < n_pairs - 1)
            return carry

        prefetch_idx(0, 0)
        prefetch_idx(1, 1)
        lax.fori_loop(0, n_pairs, pair_body, 0)

        finish(n_chunks - 1, 1, None, False)
        for h in range(2):
            wait_write(n_chunks - 1, h)

    return gather_kernel


def kernel(time, table):
    B = time.shape[0] * time.shape[1]
    NT = B // 128
    idx = time.reshape(B).astype(jnp.int32)
    o = _make_gather(B, table.shape[0])(idx, table)
    o = o.reshape(4, NT, 8, 128).transpose(0, 2, 1, 3).reshape(EMBED_DIM, B)
    return o.T
